# baseline (speedup 1.0000x reference)
import functools

import jax
import jax.numpy as jnp
from jax import lax
from jax.experimental import pallas as pl
from jax.experimental.pallas import tpu as pltpu
from jax.experimental.pallas import tpu_sc as plsc

_L = 16


@functools.lru_cache(maxsize=None)
def _build(h, w, n):
    info = plsc.get_sparse_core_info()
    nc, ns = info.num_cores, info.num_subcores
    nw = nc * ns
    per_w = n // nw
    c = min(8192, per_w)
    nchunk = per_w // c
    assert nchunk >= 4 and nchunk % 2 == 0
    nvec = c // _L
    mesh = plsc.VectorSubcoreMesh(core_axis_name="c", subcore_axis_name="s")

    @functools.partial(
        pl.kernel,
        mesh=mesh,
        out_type=jax.ShapeDtypeStruct((nw, _L), jnp.float32),
        scratch_types=[
            pltpu.VMEM((c,), jnp.float32),
            pltpu.VMEM((c,), jnp.float32),
            pltpu.VMEM((c,), jnp.float32),
            pltpu.VMEM((c,), jnp.float32),
            pltpu.VMEM((c,), jnp.float32),
            pltpu.VMEM((c,), jnp.float32),
            pltpu.VMEM((c,), jnp.int32),
            pltpu.VMEM((c,), jnp.int32),
            pltpu.VMEM((c,), jnp.int32),
            pltpu.VMEM((c,), jnp.float32),
            pltpu.VMEM((c,), jnp.float32),
            pltpu.VMEM((c,), jnp.float32),
            pltpu.VMEM((_L,), jnp.float32),
            pltpu.SemaphoreType.DMA((3,)),
            pltpu.SemaphoreType.DMA((3,)),
            pltpu.SemaphoreType.DMA((3, 4)),
        ],
    )
    def bouncer(tab, xs, ys, out, xv0, xv1, xv2, yv0, yv1, yv2,
                iv0, iv1, iv2, tv0, tv1, tv2, accv, semx, semy, semt):
        wid = lax.axis_index("s") * nc + lax.axis_index("c")
        base = wid * per_w
        xvs, yvs = (xv0, xv1, xv2), (yv0, yv1, yv2)
        ivs, tvs = (iv0, iv1, iv2), (tv0, tv1, tv2)

        def xy_copies(k, b):
            off = base + k * c
            return (
                pltpu.make_async_copy(xs.at[pl.ds(off, c)], xvs[b], semx.at[b]),
                pltpu.make_async_copy(ys.at[pl.ds(off, c)], yvs[b], semy.at[b]),
            )

        quarter = c // 4

        def gather_copies(b):
            return tuple(
                pltpu.make_async_copy(
                    tab.at[ivs[b].at[pl.ds(hf * quarter, quarter)]],
                    tvs[b].at[pl.ds(hf * quarter, quarter)],
                    semt.at[b, hf])
                for hf in range(4)
            )

        def mkidx(b):
            xv, yv, iv = xvs[b], yvs[b], ivs[b]

            def body(i, carry):
                s = pl.ds(pl.multiple_of(i * _L, _L), _L)
                xi = xv[s].astype(jnp.int32)
                yi = yv[s].astype(jnp.int32)
                iv[s] = yi * w + xi
                return carry

            lax.fori_loop(0, nvec, body, 0, unroll=8)

        def accum(b, acc):
            xv, yv, tv = xvs[b], yvs[b], tvs[b]
            mask = jnp.full((_L,), 0xFFF, jnp.int32)

            def body(i, a):
                s = pl.ds(pl.multiple_of(i * _L, _L), _L)
                p = tv[s].astype(jnp.int32)
                ty = (p & mask).astype(jnp.float32)
                tx = lax.shift_right_logical(p, 12).astype(jnp.float32)
                dx = xv[s] - tx
                dy = yv[s] - ty
                return a + dx * dx + dy * dy

            return lax.fori_loop(0, nvec, body, acc, unroll=8)

        for k in range(min(3, nchunk)):
            for cp in xy_copies(k, k % 3):
                cp.start()
        for k in range(min(2, nchunk)):
            for cp in xy_copies(k, k % 3):
                cp.wait()
            mkidx(k % 3)
            for cp in gather_copies(k % 3):
                cp.start()

        acc = jnp.zeros((_L,), jnp.float32)
        for k in range(nchunk):
            b = k % 3
            if k + 2 < nchunk:
                b2 = (k + 2) % 3
                for cp in xy_copies(k + 2, b2):
                    cp.wait()
                mkidx(b2)
                for cp in gather_copies(b2):
                    cp.start()
            for cp in gather_copies(b):
                cp.wait()
            acc = accum(b, acc)
            if k + 3 < nchunk:
                for cp in xy_copies(k + 3, b):
                    cp.start()

        accv[...] = acc
        pltpu.sync_copy(accv, out.at[wid])

    return bouncer


def kernel(dtxy, x, y):
    h, w = dtxy.shape[1], dtxy.shape[2]
    n = x.shape[0]
    tab = dtxy[1].reshape(-1) * 4096.0 + dtxy[0].reshape(-1)
    part = _build(h, w, n)(tab, x, y)
    return jnp.sum(part) / (2.0 * n)


# speedup vs baseline: 1.1140x; 1.1140x over previous
import functools

import jax
import jax.numpy as jnp
from jax import lax
from jax.experimental import pallas as pl
from jax.experimental.pallas import tpu as pltpu
from jax.experimental.pallas import tpu_sc as plsc

_L = 16


@functools.lru_cache(maxsize=None)
def _build(h, w, n):
    info = plsc.get_sparse_core_info()
    nc, ns = info.num_cores, info.num_subcores
    nw = nc * ns
    per_w = n // nw
    c = min(8192, per_w)
    nchunk = per_w // c
    assert nchunk >= 4 and nchunk % 2 == 0
    nvec = c // _L
    mesh = plsc.VectorSubcoreMesh(core_axis_name="c", subcore_axis_name="s")

    @functools.partial(
        pl.kernel,
        mesh=mesh,
        out_type=jax.ShapeDtypeStruct((nw, _L), jnp.float32),
        scratch_types=[
            pltpu.VMEM((c,), jnp.float32),
            pltpu.VMEM((c,), jnp.float32),
            pltpu.VMEM((c,), jnp.float32),
            pltpu.VMEM((c,), jnp.float32),
            pltpu.VMEM((c,), jnp.float32),
            pltpu.VMEM((c,), jnp.float32),
            pltpu.VMEM((c,), jnp.int32),
            pltpu.VMEM((c,), jnp.int32),
            pltpu.VMEM((c,), jnp.int32),
            pltpu.VMEM((c,), jnp.float32),
            pltpu.VMEM((c,), jnp.float32),
            pltpu.VMEM((c,), jnp.float32),
            pltpu.VMEM((_L,), jnp.float32),
            pltpu.SemaphoreType.DMA((3,)),
            pltpu.SemaphoreType.DMA((3,)),
            pltpu.SemaphoreType.DMA((3, 2)),
        ],
    )
    def bouncer(tab, xs, ys, out, xv0, xv1, xv2, yv0, yv1, yv2,
                iv0, iv1, iv2, tv0, tv1, tv2, accv, semx, semy, semt):
        wid = lax.axis_index("s") * nc + lax.axis_index("c")
        base = wid * per_w
        xvs, yvs = (xv0, xv1, xv2), (yv0, yv1, yv2)
        ivs, tvs = (iv0, iv1, iv2), (tv0, tv1, tv2)

        def xy_copies(k, b):
            off = base + k * c
            return (
                pltpu.make_async_copy(xs.at[pl.ds(off, c)], xvs[b], semx.at[b]),
                pltpu.make_async_copy(ys.at[pl.ds(off, c)], yvs[b], semy.at[b]),
            )

        half = c // 2

        def gather_copies(b):
            return tuple(
                pltpu.make_async_copy(
                    tab.at[ivs[b].at[pl.ds(hf * half, half)]],
                    tvs[b].at[pl.ds(hf * half, half)],
                    semt.at[b, hf])
                for hf in range(2)
            )

        def mkidx(b):
            xv, yv, iv = xvs[b], yvs[b], ivs[b]

            def body(i, carry):
                s = pl.ds(pl.multiple_of(i * _L, _L), _L)
                xi = xv[s].astype(jnp.int32)
                yi = yv[s].astype(jnp.int32)
                iv[s] = yi * w + xi
                return carry

            lax.fori_loop(0, nvec, body, 0, unroll=8)

        def accum(b, acc):
            xv, yv, tv = xvs[b], yvs[b], tvs[b]
            mask = jnp.full((_L,), 0xFFF, jnp.int32)

            def body(i, a):
                s = pl.ds(pl.multiple_of(i * _L, _L), _L)
                p = tv[s].astype(jnp.int32)
                ty = (p & mask).astype(jnp.float32)
                tx = lax.shift_right_logical(p, 12).astype(jnp.float32)
                dx = xv[s] - tx
                dy = yv[s] - ty
                return a + dx * dx + dy * dy

            return lax.fori_loop(0, nvec, body, acc, unroll=8)

        for k in range(min(3, nchunk)):
            for cp in xy_copies(k, k % 3):
                cp.start()
        for k in range(min(2, nchunk)):
            for cp in xy_copies(k, k % 3):
                cp.wait()
            mkidx(k % 3)
            for cp in gather_copies(k % 3):
                cp.start()

        acc = jnp.zeros((_L,), jnp.float32)
        for k in range(nchunk):
            b = k % 3
            if k + 2 < nchunk:
                b2 = (k + 2) % 3
                for cp in xy_copies(k + 2, b2):
                    cp.wait()
                mkidx(b2)
                for cp in gather_copies(b2):
                    cp.start()
            for cp in gather_copies(b):
                cp.wait()
            acc = accum(b, acc)
            if k + 3 < nchunk:
                for cp in xy_copies(k + 3, b):
                    cp.start()

        accv[...] = acc
        pltpu.sync_copy(accv, out.at[wid])

    return bouncer


def kernel(dtxy, x, y):
    h, w = dtxy.shape[1], dtxy.shape[2]
    n = x.shape[0]
    tab = dtxy[1].reshape(-1) * 4096.0 + dtxy[0].reshape(-1)
    part = _build(h, w, n)(tab, x, y)
    return jnp.sum(part) / (2.0 * n)
